# Initial kernel scaffold; baseline (speedup 1.0000x reference)
import functools

import jax
import jax.numpy as jnp
from jax import lax
from jax.experimental import pallas as pl
from jax.experimental.pallas import tpu as pltpu
from jax.experimental.pallas import tpu_sc as plsc

_N = 10000
_D = 128
_NC = 2
_NS = 16
_NW = _NC * _NS
_CH = 128
_NCH = 80
_EPAD = _NW * _NCH * _CH
_TRASH = _N
_ACC_R = 10240
_HW = 16
_RB = 1000


def _sc_mesh():
    return plsc.VectorSubcoreMesh(
        core_axis_name="c", subcore_axis_name="s",
        num_cores=_NC, num_subcores=_NS)


@functools.partial(
    pl.kernel,
    out_type=jax.ShapeDtypeStruct((_NC, _ACC_R, _HW), jnp.float32),
    mesh=_sc_mesh(),
    scratch_types=[
        pltpu.VMEM((_NCH, _CH), jnp.int32),
        pltpu.VMEM((_CH, _HW), jnp.float32),
        pltpu.VMEM((64, _HW), jnp.float32),
        pltpu.VMEM_SHARED((_ACC_R, _HW), jnp.float32),
    ],
)
def _sc_degree(dst_hbm, out_hbm, dst_v, ones_v, zero_v, hist):
    c = lax.axis_index("c")
    s = lax.axis_index("s")
    wid = c * _NS + s
    ov = jnp.ones((16,), jnp.float32)
    zv = jnp.zeros((16,), jnp.float32)
    for i in range(_CH):
        ones_v[i, pl.ds(0, 16)] = ov
    for i in range(64):
        zero_v[i, pl.ds(0, 16)] = zv
    rows_per = _ACC_R // _NS
    base = s * rows_per

    def zbody(k, carry):
        pltpu.sync_copy(zero_v, hist.at[pl.ds(base + k * 64, 64)])
        return carry

    lax.fori_loop(0, rows_per // 64, zbody, 0)
    plsc.subcore_barrier()

    pltpu.sync_copy(dst_hbm.at[wid], dst_v)

    def body(jj, carry):
        pltpu.sync_copy(ones_v, hist.at[dst_v.at[jj]], add=True)
        return carry

    lax.fori_loop(0, _NCH, body, 0)
    plsc.subcore_barrier()
    pltpu.sync_copy(hist.at[pl.ds(base, rows_per)],
                    out_hbm.at[c, pl.ds(base, rows_per)])


@functools.partial(
    pl.kernel,
    out_type=jax.ShapeDtypeStruct((_NC, _N, _D), jnp.float32),
    mesh=_sc_mesh(),
    scratch_types=[
        pltpu.VMEM((_NCH + 2, _CH), jnp.int32),
        pltpu.VMEM((_NCH, _CH), jnp.int32),
        pltpu.VMEM((2, _CH, _D), jnp.float32),
        pltpu.VMEM((64, _D), jnp.float32),
        pltpu.VMEM_SHARED((_ACC_R, _D), jnp.float32),
        pltpu.SemaphoreType.DMA,
        pltpu.SemaphoreType.DMA,
    ],
)
def _sc_scatter(hs_hbm, src_hbm, dst_hbm, out_hbm,
                src_v, dst_v, rows_v, zero_v, acc, sem0, sem1):
    c = lax.axis_index("c")
    s = lax.axis_index("s")
    wid = c * _NS + s
    zv = jnp.zeros((16,), jnp.float32)
    for i in range(64):
        for j in range(_D // 16):
            zero_v[i, pl.ds(j * 16, 16)] = zv
    rows_per = _ACC_R // _NS
    base = s * rows_per

    def zbody(k, carry):
        pltpu.sync_copy(zero_v, acc.at[pl.ds(base + k * 64, 64)])
        return carry

    lax.fori_loop(0, rows_per // 64, zbody, 0)
    plsc.subcore_barrier()

    pltpu.sync_copy(src_hbm.at[wid], src_v)
    pltpu.sync_copy(dst_hbm.at[wid], dst_v)

    sems = (sem0, sem1)
    for b in range(2):
        pltpu.async_copy(hs_hbm.at[src_v.at[b]], rows_v.at[b], sems[b])

    def body(t, carry):
        for b in range(2):
            jj = t * 2 + b
            pltpu.make_async_copy(
                hs_hbm.at[pl.ds(0, _CH)], rows_v.at[b], sems[b]).wait()
            pltpu.sync_copy(rows_v.at[b], acc.at[dst_v.at[jj]], add=True)
            pltpu.async_copy(hs_hbm.at[src_v.at[jj + 2]], rows_v.at[b],
                             sems[b])
        return carry

    lax.fori_loop(0, _NCH // 2, body, 0)
    for b in range(2):
        pltpu.make_async_copy(
            hs_hbm.at[pl.ds(0, _CH)], rows_v.at[b], sems[b]).wait()
    plsc.subcore_barrier()

    out_rows = _N // _NS
    ob = s * out_rows
    pltpu.sync_copy(acc.at[pl.ds(ob, out_rows)],
                    out_hbm.at[c, pl.ds(ob, out_rows)])


def _dinv_from_hist(h_ref):
    cnt = h_ref[0, :, 0:1] + h_ref[1, :, 0:1]
    return lax.rsqrt(cnt + 1.0)


def _tc_first(x, W1, hist):
    def body(x_ref, w_ref, h_ref, o_ref):
        dinv = _dinv_from_hist(h_ref)
        h = jnp.dot(x_ref[...], w_ref[...],
                    preferred_element_type=jnp.float32)
        o_ref[...] = h * dinv

    return pl.pallas_call(
        body,
        grid=(_N // _RB,),
        in_specs=[
            pl.BlockSpec((_RB, _D), lambda i: (i, 0)),
            pl.BlockSpec((_D, _D), lambda i: (0, 0)),
            pl.BlockSpec((_NC, _RB, _HW), lambda i: (0, i, 0)),
        ],
        out_specs=pl.BlockSpec((_RB, _D), lambda i: (i, 0)),
        out_shape=jax.ShapeDtypeStruct((_N, _D), jnp.float32),
    )(x, W1, hist)


def _tc_mid(acc, hs, hist, b1, W2):
    def body(a_ref, hs_ref, h_ref, b_ref, w_ref, o_ref):
        dinv = _dinv_from_hist(h_ref)
        z = (a_ref[0] + a_ref[1] + hs_ref[...]) * dinv + b_ref[...]
        z = jnp.maximum(z, 0.0)
        o_ref[...] = jnp.dot(z, w_ref[...],
                             preferred_element_type=jnp.float32) * dinv

    return pl.pallas_call(
        body,
        grid=(_N // _RB,),
        in_specs=[
            pl.BlockSpec((_NC, _RB, _D), lambda i: (0, i, 0)),
            pl.BlockSpec((_RB, _D), lambda i: (i, 0)),
            pl.BlockSpec((_NC, _RB, _HW), lambda i: (0, i, 0)),
            pl.BlockSpec((1, _D), lambda i: (0, 0)),
            pl.BlockSpec((_D, _D), lambda i: (0, 0)),
        ],
        out_specs=pl.BlockSpec((_RB, _D), lambda i: (i, 0)),
        out_shape=jax.ShapeDtypeStruct((_N, _D), jnp.float32),
    )(acc, hs, hist, b1, W2)


def _tc_last(acc, hs, hist, b2):
    def body(a_ref, hs_ref, h_ref, b_ref, o_ref):
        dinv = _dinv_from_hist(h_ref)
        z = (a_ref[0] + a_ref[1] + hs_ref[...]) * dinv + b_ref[...]
        o_ref[...] = jnp.maximum(z, 0.0)

    return pl.pallas_call(
        body,
        grid=(_N // _RB,),
        in_specs=[
            pl.BlockSpec((_NC, _RB, _D), lambda i: (0, i, 0)),
            pl.BlockSpec((_RB, _D), lambda i: (i, 0)),
            pl.BlockSpec((_NC, _RB, _HW), lambda i: (0, i, 0)),
            pl.BlockSpec((1, _D), lambda i: (0, 0)),
        ],
        out_specs=pl.BlockSpec((_RB, _D), lambda i: (i, 0)),
        out_shape=jax.ShapeDtypeStruct((_N, _D), jnp.float32),
    )(acc, hs, hist, b2)


def kernel(x, edge_index, W1, b1, W2, b2):
    src = edge_index[0]
    dst = edge_index[1]
    e = src.shape[0]
    pad = _EPAD - e
    src_p = jnp.concatenate([src, jnp.zeros((pad,), jnp.int32)])
    dst_p = jnp.concatenate([dst, jnp.full((pad,), _TRASH, jnp.int32)])
    src3 = src_p.reshape(_NW, _NCH, _CH)
    src3 = jnp.concatenate(
        [src3, jnp.zeros((_NW, 2, _CH), jnp.int32)], axis=1)
    dst3 = dst_p.reshape(_NW, _NCH, _CH)

    hist_full = _sc_degree(dst3)
    hist = hist_full[:, :_N, :]
    b1r = b1.reshape(1, _D)
    b2r = b2.reshape(1, _D)

    h1s = _tc_first(x, W1, hist)
    acc1 = _sc_scatter(h1s, src3, dst3)
    h2s = _tc_mid(acc1, h1s, hist, b1r, W2)
    acc2 = _sc_scatter(h2s, src3, dst3)
    out = _tc_last(acc2, h2s, hist, b2r)
    return out


# speedup vs baseline: 5.7230x; 5.7230x over previous
import functools

import jax
import jax.numpy as jnp
from jax import lax
from jax.experimental import pallas as pl
from jax.experimental.pallas import tpu as pltpu
from jax.experimental.pallas import tpu_sc as plsc

_N = 10000
_D = 128
_NC = 2
_NS = 16
_NW = _NC * _NS
_CH = 128
_NCH = 80
_PCH = 40
_EPAD = _NW * _NCH * _CH
_TRASH = _N
_ACC_R = 10240
_HG = 80
_RB = 1000


def _sc_mesh():
    return plsc.VectorSubcoreMesh(
        core_axis_name="c", subcore_axis_name="s",
        num_cores=_NC, num_subcores=_NS)


@functools.partial(
    pl.kernel,
    out_type=jax.ShapeDtypeStruct((_NC, _ACC_R, _D), jnp.float32),
    mesh=_sc_mesh(),
    scratch_types=[
        pltpu.VMEM((_NCH, _CH), jnp.int32),
        pltpu.VMEM((16, _D), jnp.float32),
        pltpu.VMEM((_CH, _D), jnp.float32),
        pltpu.VMEM_SHARED((_ACC_R, _D), jnp.float32),
    ],
)
def _sc_degree(dst_hbm, out_hbm, dst_v, zeros_v, ones_v, acc):
    c = lax.axis_index("c")
    s = lax.axis_index("s")
    wid = c * _NS + s
    zv = jnp.zeros((16,), jnp.float32)
    ov = jnp.ones((16,), jnp.float32)
    for i in range(16):
        for j in range(_D // 16):
            zeros_v[i, pl.ds(j * 16, 16)] = zv
    for i in range(_CH):
        for j in range(_D // 16):
            ones_v[i, pl.ds(j * 16, 16)] = ov
    rows_per = _ACC_R // _NS
    base = s * rows_per

    def zbody(k, carry):
        pltpu.sync_copy(zeros_v, acc.at[pl.ds(base + k * 16, 16)])
        return carry

    lax.fori_loop(0, rows_per // 16, zbody, 0)
    plsc.subcore_barrier()

    pltpu.sync_copy(dst_hbm.at[wid], dst_v)

    def body(jj, carry):
        pltpu.sync_copy(ones_v, acc.at[dst_v.at[jj]], add=True)
        return carry

    lax.fori_loop(0, _NCH, body, 0)
    plsc.subcore_barrier()
    pltpu.sync_copy(acc.at[pl.ds(base, rows_per)],
                    out_hbm.at[c, pl.ds(base, rows_per)])


@functools.partial(
    pl.kernel,
    out_type=jax.ShapeDtypeStruct((_NC, _ACC_R, _D), jnp.float32),
    mesh=_sc_mesh(),
    scratch_types=[
        pltpu.VMEM((_PCH + 8, _CH), jnp.int32),
        pltpu.VMEM((_PCH, _CH), jnp.int32),
        pltpu.VMEM((2, _CH, _D), jnp.float32),
        pltpu.VMEM_SHARED((_ACC_R, _D), jnp.float32),
        pltpu.SemaphoreType.DMA,
        pltpu.SemaphoreType.DMA,
    ],
)
def _sc_scatter(hs_hbm, src_hbm, dst_hbm, out_hbm,
                src_v, dst_v, rows_v, acc, sem0, sem1):
    c = lax.axis_index("c")
    s = lax.axis_index("s")
    wid = c * _NS + s
    zv = jnp.zeros((16,), jnp.float32)
    for i in range(16):
        for j in range(_D // 16):
            rows_v[0, i, pl.ds(j * 16, 16)] = zv
    rows_per = _ACC_R // _NS
    base = s * rows_per

    def zbody(k, carry):
        pltpu.sync_copy(rows_v.at[0, pl.ds(0, 16)],
                        acc.at[pl.ds(base + k * 16, 16)])
        return carry

    lax.fori_loop(0, rows_per // 16, zbody, 0)
    plsc.subcore_barrier()

    sems = (sem0, sem1)
    for p in range(_NCH // _PCH):
        pltpu.sync_copy(src_hbm.at[wid, pl.ds(p * _PCH, _PCH + 8)], src_v)
        pltpu.sync_copy(dst_hbm.at[wid, pl.ds(p * _PCH, _PCH)], dst_v)
        for b in range(2):
            pltpu.async_copy(hs_hbm.at[src_v.at[b]], rows_v.at[b], sems[b])

        def body(t, carry):
            for b in range(2):
                jj = t * 2 + b
                pltpu.make_async_copy(
                    hs_hbm.at[pl.ds(0, _CH)], rows_v.at[b], sems[b]).wait()
                pltpu.sync_copy(rows_v.at[b], acc.at[dst_v.at[jj]], add=True)
                pltpu.async_copy(hs_hbm.at[src_v.at[jj + 2]], rows_v.at[b],
                                 sems[b])
            return carry

        lax.fori_loop(0, _PCH // 2, body, 0)
        for b in range(2):
            pltpu.make_async_copy(
                hs_hbm.at[pl.ds(0, _CH)], rows_v.at[b], sems[b]).wait()
    plsc.subcore_barrier()

    pltpu.sync_copy(acc.at[pl.ds(base, rows_per)],
                    out_hbm.at[c, pl.ds(base, rows_per)])


def _tc_dinv(deg):
    def body(a_ref, o_ref):
        cnt = a_ref[0, :, 0:1] + a_ref[1, :, 0:1]
        o_ref[...] = lax.rsqrt(cnt + 1.0)

    return pl.pallas_call(
        body,
        grid=(_ACC_R // 1024,),
        in_specs=[pl.BlockSpec((_NC, 1024, _D), lambda i: (0, i, 0))],
        out_specs=pl.BlockSpec((1024, 1), lambda i: (i, 0)),
        out_shape=jax.ShapeDtypeStruct((_ACC_R, 1), jnp.float32),
    )(deg)


def _tc_first(x, W1):
    def body(x_ref, w_ref, o_ref):
        o_ref[...] = jnp.dot(x_ref[...], w_ref[...],
                             preferred_element_type=jnp.float32)

    return pl.pallas_call(
        body,
        grid=(_N // _RB,),
        in_specs=[
            pl.BlockSpec((_RB, _D), lambda i: (i, 0)),
            pl.BlockSpec((_D, _D), lambda i: (0, 0)),
        ],
        out_specs=pl.BlockSpec((_RB, _D), lambda i: (i, 0)),
        out_shape=jax.ShapeDtypeStruct((_N, _D), jnp.float32),
    )(x, W1)


def _tc_scale(h, dinv):
    def body(h_ref, d_ref, o_ref):
        o_ref[...] = h_ref[...] * d_ref[...]

    return pl.pallas_call(
        body,
        grid=(_N // _RB,),
        in_specs=[
            pl.BlockSpec((_RB, _D), lambda i: (i, 0)),
            pl.BlockSpec((_RB, 1), lambda i: (i, 0)),
        ],
        out_specs=pl.BlockSpec((_RB, _D), lambda i: (i, 0)),
        out_shape=jax.ShapeDtypeStruct((_N, _D), jnp.float32),
    )(h, dinv)


def _tc_mid(acc, hs, dinv, b1, W2):
    def body(a_ref, hs_ref, d_ref, b_ref, w_ref, o_ref):
        z = (a_ref[0] + a_ref[1] + hs_ref[...]) * d_ref[...] + b_ref[...]
        z = jnp.maximum(z, 0.0)
        o_ref[...] = jnp.dot(z, w_ref[...],
                             preferred_element_type=jnp.float32) * d_ref[...]

    return pl.pallas_call(
        body,
        grid=(_N // _RB,),
        in_specs=[
            pl.BlockSpec((_NC, _RB, _D), lambda i: (0, i, 0)),
            pl.BlockSpec((_RB, _D), lambda i: (i, 0)),
            pl.BlockSpec((_RB, 1), lambda i: (i, 0)),
            pl.BlockSpec((1, _D), lambda i: (0, 0)),
            pl.BlockSpec((_D, _D), lambda i: (0, 0)),
        ],
        out_specs=pl.BlockSpec((_RB, _D), lambda i: (i, 0)),
        out_shape=jax.ShapeDtypeStruct((_N, _D), jnp.float32),
    )(acc, hs, dinv, b1, W2)


def _tc_last(acc, hs, dinv, b2):
    def body(a_ref, hs_ref, d_ref, b_ref, o_ref):
        z = (a_ref[0] + a_ref[1] + hs_ref[...]) * d_ref[...] + b_ref[...]
        o_ref[...] = jnp.maximum(z, 0.0)

    return pl.pallas_call(
        body,
        grid=(_N // _RB,),
        in_specs=[
            pl.BlockSpec((_NC, _RB, _D), lambda i: (0, i, 0)),
            pl.BlockSpec((_RB, _D), lambda i: (i, 0)),
            pl.BlockSpec((_RB, 1), lambda i: (i, 0)),
            pl.BlockSpec((1, _D), lambda i: (0, 0)),
        ],
        out_specs=pl.BlockSpec((_RB, _D), lambda i: (i, 0)),
        out_shape=jax.ShapeDtypeStruct((_N, _D), jnp.float32),
    )(acc, hs, dinv, b2)


def kernel(x, edge_index, W1, b1, W2, b2):
    src = edge_index[0]
    dst = edge_index[1]
    e = src.shape[0]
    pad = _EPAD - e
    src_p = jnp.concatenate([src, jnp.zeros((pad,), jnp.int32)])
    dst_p = jnp.concatenate([dst, jnp.full((pad,), _TRASH, jnp.int32)])
    src3 = src_p.reshape(_NW, _NCH, _CH)
    src3 = jnp.concatenate(
        [src3, jnp.zeros((_NW, 8, _CH), jnp.int32)], axis=1)
    dst3 = dst_p.reshape(_NW, _NCH, _CH)

    deg = _sc_degree(dst3)
    h1 = _tc_first(x, W1)
    dinv = _tc_dinv(deg)[:_N]
    b1r = b1.reshape(1, _D)
    b2r = b2.reshape(1, _D)

    h1s = _tc_scale(h1, dinv)
    acc1 = _sc_scatter(h1s, src3, dst3)
    h2s = _tc_mid(acc1, h1s, dinv, b1r, W2)
    acc2 = _sc_scatter(h2s, src3, dst3)
    out = _tc_last(acc2, h2s, dinv, b2r)
    return out
